# Initial kernel scaffold; baseline (speedup 1.0000x reference)
import jax
import jax.numpy as jnp
from jax.experimental import pallas as pl


def kernel(x, cluster_assignments, cluster_centers):
    raise NotImplementedError("write your pallas kernel here")


# speedup vs baseline: 6.9286x; 6.9286x over previous
import jax
import jax.numpy as jnp
from jax.experimental import pallas as pl

_K = 64
_N = 50000
_D = 256
_B = 2000
_NB = _N // _B


def _loss_block(x_ref, a_ref, c_ref, out_ref):
    i = pl.program_id(0)
    x = x_ref[...]
    a = a_ref[0]
    c = c_ref[...]

    xs = jnp.sum(x * x)
    scores = jax.lax.dot_general(
        c, x, (((1,), (1,)), ((), ())), preferred_element_type=jnp.float32)
    cn = jnp.sum(c * c, axis=1, keepdims=True)
    m = cn - 2.0 * scores
    row = jax.lax.broadcasted_iota(jnp.int32, (_K, _B), 0)
    oh = row == a
    s = jax.lax.broadcast(xs + jnp.sum(jnp.where(oh, m, 0.0)), (1, 1))

    @pl.when(i == 0)
    def _():
        out_ref[...] = s

    @pl.when(i != 0)
    def _():
        out_ref[...] += s


def kernel(x, cluster_assignments, cluster_centers):
    a3 = cluster_assignments.reshape(_NB, 1, _B)
    out = pl.pallas_call(
        _loss_block,
        grid=(_NB,),
        in_specs=[
            pl.BlockSpec((_B, _D), lambda i: (i, 0)),
            pl.BlockSpec((1, 1, _B), lambda i: (i, 0, 0)),
            pl.BlockSpec((_K, _D), lambda i: (0, 0)),
        ],
        out_specs=pl.BlockSpec((1, 1), lambda i: (0, 0)),
        out_shape=jax.ShapeDtypeStruct((1, 1), jnp.float32),
    )(x, a3, cluster_centers)
    return out[0, 0]


# speedup vs baseline: 9.1987x; 1.3276x over previous
import jax
import jax.numpy as jnp
from jax.experimental import pallas as pl

_K = 64
_N = 50000
_D = 256
_B = 5000
_NB = _N // _B


def _loss_block(x_ref, a_ref, c_ref, out_ref):
    i = pl.program_id(0)
    x = x_ref[...]
    a = a_ref[0]
    c = c_ref[...]

    xs = jnp.sum(x * x)
    scores = jax.lax.dot_general(
        c, x, (((1,), (1,)), ((), ())), preferred_element_type=jnp.float32)
    cn = jnp.sum(c * c, axis=1, keepdims=True)
    m = cn - 2.0 * scores
    row = jax.lax.broadcasted_iota(jnp.int32, (_K, _B), 0)
    oh = row == a
    s = jax.lax.broadcast(xs + jnp.sum(jnp.where(oh, m, 0.0)), (1, 1))

    @pl.when(i == 0)
    def _():
        out_ref[...] = s

    @pl.when(i != 0)
    def _():
        out_ref[...] += s


def kernel(x, cluster_assignments, cluster_centers):
    a3 = cluster_assignments.reshape(_NB, 1, _B)
    out = pl.pallas_call(
        _loss_block,
        grid=(_NB,),
        in_specs=[
            pl.BlockSpec((_B, _D), lambda i: (i, 0)),
            pl.BlockSpec((1, 1, _B), lambda i: (i, 0, 0)),
            pl.BlockSpec((_K, _D), lambda i: (0, 0)),
        ],
        out_specs=pl.BlockSpec((1, 1), lambda i: (0, 0)),
        out_shape=jax.ShapeDtypeStruct((1, 1), jnp.float32),
    )(x, a3, cluster_centers)
    return out[0, 0]


# speedup vs baseline: 9.6033x; 1.0440x over previous
import jax
import jax.numpy as jnp
from jax.experimental import pallas as pl

_K = 64
_N = 50000
_D = 256
_B = 10000
_NB = _N // _B


def _loss_block(x_ref, a_ref, c_ref, out_ref):
    i = pl.program_id(0)
    x = x_ref[...]
    a = a_ref[0]
    c = c_ref[...]

    xs = jnp.sum(x * x)
    scores = jax.lax.dot_general(
        c, x, (((1,), (1,)), ((), ())), preferred_element_type=jnp.float32)
    cn = jnp.sum(c * c, axis=1, keepdims=True)
    m = cn - 2.0 * scores
    row = jax.lax.broadcasted_iota(jnp.int32, (_K, _B), 0)
    oh = row == a
    s = jax.lax.broadcast(xs + jnp.sum(jnp.where(oh, m, 0.0)), (1, 1))

    @pl.when(i == 0)
    def _():
        out_ref[...] = s

    @pl.when(i != 0)
    def _():
        out_ref[...] += s


def kernel(x, cluster_assignments, cluster_centers):
    a3 = cluster_assignments.reshape(_NB, 1, _B)
    out = pl.pallas_call(
        _loss_block,
        grid=(_NB,),
        in_specs=[
            pl.BlockSpec((_B, _D), lambda i: (i, 0)),
            pl.BlockSpec((1, 1, _B), lambda i: (i, 0, 0)),
            pl.BlockSpec((_K, _D), lambda i: (0, 0)),
        ],
        out_specs=pl.BlockSpec((1, 1), lambda i: (0, 0)),
        out_shape=jax.ShapeDtypeStruct((1, 1), jnp.float32),
    )(x, a3, cluster_centers)
    return out[0, 0]
